# baseline (speedup 1.0000x reference)
import functools

import jax
import jax.numpy as jnp
from jax import lax
from jax.experimental import pallas as pl
from jax.experimental.pallas import tpu as pltpu
from jax.experimental.pallas import tpu_sc as plsc

B, S = 4096, 50
FEAT, HID = 128, 128
N = B * S
EPS = 1e-12

NC, NS = 2, 16
NW = NC * NS
NSPLIT = 4
NHALF = N // NSPLIT
ROWS_PER_W = NHALF // NW
R = 32
NCHUNK = ROWS_PER_W // R
NPAIR = NCHUNK // 2


def _pack_bf16_lohi(t):
    u = lax.bitcast_convert_type(t, jnp.uint32)
    bf = (u + jnp.uint32(0x7FFF) + ((u >> 16) & jnp.uint32(1))) >> 16
    v = bf.reshape(t.shape[0], 4, 2, 16)
    packed = (v[:, :, 1, :] << 16) | v[:, :, 0, :]
    return lax.bitcast_convert_type(packed.reshape(t.shape[0] * 64), jnp.int32)


POSV, HOPV = 1000, 100


def _sc_gather_sum_build(h):
    mesh = plsc.VectorSubcoreMesh(core_axis_name="c", subcore_axis_name="s")

    @functools.partial(
        pl.kernel,
        mesh=mesh,
        compiler_params=pltpu.CompilerParams(needs_layout_passes=False),
        out_type=jax.ShapeDtypeStruct((NHALF, HID), jnp.float32),
        scratch_types=[
            pltpu.VMEM((ROWS_PER_W,), jnp.int32),
            pltpu.VMEM((ROWS_PER_W,), jnp.int32),
            pltpu.VMEM((ROWS_PER_W,), jnp.int32),
            pltpu.VMEM((POSV * HID // 2,), jnp.int32),
            pltpu.VMEM((HOPV * HID // 2,), jnp.int32),
            pltpu.VMEM((R, HID), jnp.float32),
            pltpu.VMEM((R, HID), jnp.float32),
            pltpu.VMEM((R, HID), jnp.float32),
            pltpu.VMEM((R, HID), jnp.float32),
            pltpu.SemaphoreType.DMA,
            pltpu.SemaphoreType.DMA,
            pltpu.SemaphoreType.DMA,
            pltpu.SemaphoreType.DMA,
        ],
    )
    def sc_gather_sum(wl_ids, pos_ids, hop_ids, wl_t, pos_p, hop_p, out,
                      iwl, ipos, ihop, posv, hopv,
                      gwl0, gwl1, ob0, ob1,
                      gsem0, gsem1, wsem0, wsem1):
        gwl = (gwl0, gwl1)
        ob = (ob0, ob1)
        gsem = (gsem0, gsem1)
        wsem = (wsem0, wsem1)

        wid = lax.axis_index("s") * NC + lax.axis_index("c")
        wbase = pl.multiple_of(wid * ROWS_PER_W, ROWS_PER_W)
        gbase = pl.multiple_of(h * NHALF + wid * ROWS_PER_W, ROWS_PER_W)

        pltpu.sync_copy(wl_ids.at[pl.ds(gbase, ROWS_PER_W)], iwl)
        pltpu.sync_copy(pos_ids.at[pl.ds(gbase, ROWS_PER_W)], ipos)
        pltpu.sync_copy(hop_ids.at[pl.ds(gbase, ROWS_PER_W)], ihop)
        pltpu.sync_copy(pos_p, posv)
        pltpu.sync_copy(hop_p, hopv)

        def start_gather(s, off):
            pltpu.async_copy(wl_t.at[iwl.at[pl.ds(off, R)]], gwl[s], gsem[s])

        def wait_gather(s):
            pltpu.make_async_copy(wl_t.at[pl.ds(0, R)], gwl[s], gsem[s]).wait()

        def add_pass(s, off):
            gb = gwl[s]
            obs = ob[s]
            himask = jnp.int32(-65536)

            def grp_body(k, gcarry):
                rb = k * 16
                pid16 = ipos[pl.ds(off + rb, 16)]
                hid16 = ihop[pl.ds(off + rb, 16)]
                for j in range(16):
                    r = rb + j
                    selj = jnp.full((16,), j, jnp.int32)
                    prow = pid16.at[selj].get(mode="promise_in_bounds") << 6
                    hrow = hid16.at[selj].get(mode="promise_in_bounds") << 6
                    for cg in range(4):
                        colv = lax.iota(jnp.int32, 16) + (16 * cg)
                        pp = plsc.load_gather(posv, [prow + colv])
                        hp = plsc.load_gather(hopv, [hrow + colv])
                        plo = plsc.bitcast(pp << 16, jnp.float32)
                        phi = plsc.bitcast(pp & himask, jnp.float32)
                        hlo = plsc.bitcast(hp << 16, jnp.float32)
                        hhi = plsc.bitcast(hp & himask, jnp.float32)
                        slo = pl.ds(32 * cg, 16)
                        shi = pl.ds(32 * cg + 16, 16)
                        obs[r, slo] = gb[r, slo] + plo + hlo
                        obs[r, shi] = gb[r, shi] + phi + hhi
                return gcarry

            lax.fori_loop(0, R // 16, grp_body, 0)

        def start_write(s, off):
            pltpu.async_copy(ob[s], out.at[pl.ds(wbase + off, R)], wsem[s])

        def wait_write(s):
            pltpu.make_async_copy(ob[s], out.at[pl.ds(0, R)], wsem[s]).wait()

        start_gather(0, 0)
        start_gather(1, R)

        def pair_body(p, carry):
            off0 = p * (2 * R)

            @pl.when(p > 0)
            def _():
                wait_write(0)
            wait_gather(0)
            add_pass(0, off0)
            start_write(0, off0)

            @pl.when(p < NPAIR - 1)
            def _():
                start_gather(0, off0 + 2 * R)

            @pl.when(p > 0)
            def _():
                wait_write(1)
            wait_gather(1)
            add_pass(1, off0 + R)
            start_write(1, off0 + R)

            @pl.when(p < NPAIR - 1)
            def _():
                start_gather(1, off0 + 3 * R)

            return carry

        lax.fori_loop(0, NPAIR, pair_body, 0)
        wait_write(0)
        wait_write(1)

    return sc_gather_sum


_sc_gather_sum = tuple(_sc_gather_sum_build(h) for h in range(NSPLIT))


TBB = 64
TROWS = TBB * S
NBLK = B // TBB
NBLK_H = NBLK // NSPLIT


def _tc_finish_body(raw_ref, w_ref, b_ref, g_ref, gamma_ref, beta_ref,
                    part_ref, out_ref):
    del part_ref
    x3 = raw_ref[...]
    x = jnp.dot(x3.reshape(TROWS, FEAT), w_ref[...],
                preferred_element_type=jnp.float32)
    x = x + b_ref[...] + g_ref[...]
    mu = jnp.mean(x, axis=1, keepdims=True)
    d = x - mu
    var = jnp.mean(d * d, axis=1, keepdims=True)
    y = d * lax.rsqrt(var + EPS)
    y = y * gamma_ref[...] + beta_ref[...]
    out_ref[...] = y.reshape(TBB, S, HID)


def _tc_finish_build(h):
    off = h * NBLK_H
    return pl.pallas_call(
        _tc_finish_body,
        grid=(NBLK_H,),
        in_specs=[
            pl.BlockSpec((TBB, S, FEAT), lambda i: (i + off, 0, 0)),
            pl.BlockSpec((FEAT, HID), lambda i: (0, 0)),
            pl.BlockSpec((1, HID), lambda i: (0, 0)),
            pl.BlockSpec((TROWS, HID), lambda i: (i, 0)),
            pl.BlockSpec((1, HID), lambda i: (0, 0)),
            pl.BlockSpec((1, HID), lambda i: (0, 0)),
            pl.BlockSpec(memory_space=pl.ANY),
        ],
        out_specs=pl.BlockSpec((TBB, S, HID), lambda i: (i + off, 0, 0)),
        out_shape=jax.ShapeDtypeStruct((B, S, HID), jnp.float32),
        input_output_aliases={6: 0},
    )


_tc_finish = tuple(_tc_finish_build(h) for h in range(NSPLIT))


def kernel(raw_features, wl_role_ids, init_pos_ids, hop_dis_ids, W, b,
           wl_table, pos_table, hop_table, ln_gamma, ln_beta):
    wl_ids = wl_role_ids.reshape(N)
    pos_ids = init_pos_ids.reshape(N)
    hop_ids = hop_dis_ids.reshape(N)
    pos_p = _pack_bf16_lohi(pos_table)
    hop_p = _pack_bf16_lohi(hop_table)
    gs = [f(wl_ids, pos_ids, hop_ids, wl_table, pos_p, hop_p)
          for f in _sc_gather_sum]
    b2 = b.reshape(1, HID)
    gm = ln_gamma.reshape(1, HID)
    bt = ln_beta.reshape(1, HID)
    out = jnp.zeros((B, S, HID), jnp.float32)
    for h in range(NSPLIT):
        out = _tc_finish[h](raw_features, W, b2, gs[h], gm, bt, out)
    return out


# speedup vs baseline: 1.1347x; 1.1347x over previous
import functools

import jax
import jax.numpy as jnp
from jax import lax
from jax.experimental import pallas as pl
from jax.experimental.pallas import tpu as pltpu
from jax.experimental.pallas import tpu_sc as plsc

B, S = 4096, 50
FEAT, HID = 128, 128
N = B * S
EPS = 1e-12

NC, NS = 2, 16
NW = NC * NS
NSPLIT = 2
NHALF = N // NSPLIT
ROWS_PER_W = NHALF // NW
R = 80
NCHUNK = ROWS_PER_W // R
NPAIR = NCHUNK // 2


def _pack_bf16_lohi(t):
    u = lax.bitcast_convert_type(t, jnp.uint32)
    bf = (u + jnp.uint32(0x7FFF) + ((u >> 16) & jnp.uint32(1))) >> 16
    v = bf.reshape(t.shape[0], 4, 2, 16)
    packed = (v[:, :, 1, :] << 16) | v[:, :, 0, :]
    return lax.bitcast_convert_type(packed.reshape(t.shape[0] * 64), jnp.int32)


POSV, HOPV = 1000, 100


def _sc_gather_sum_build(h):
    mesh = plsc.VectorSubcoreMesh(core_axis_name="c", subcore_axis_name="s")

    @functools.partial(
        pl.kernel,
        mesh=mesh,
        compiler_params=pltpu.CompilerParams(needs_layout_passes=False),
        out_type=jax.ShapeDtypeStruct((NHALF, HID), jnp.float32),
        scratch_types=[
            pltpu.VMEM((ROWS_PER_W,), jnp.int32),
            pltpu.VMEM((ROWS_PER_W,), jnp.int32),
            pltpu.VMEM((ROWS_PER_W,), jnp.int32),
            pltpu.VMEM((POSV * HID // 2,), jnp.int32),
            pltpu.VMEM((HOPV * HID // 2,), jnp.int32),
            pltpu.VMEM((R, HID), jnp.float32),
            pltpu.VMEM((R, HID), jnp.float32),
            pltpu.VMEM((R, HID), jnp.float32),
            pltpu.VMEM((R, HID), jnp.float32),
            pltpu.SemaphoreType.DMA,
            pltpu.SemaphoreType.DMA,
            pltpu.SemaphoreType.DMA,
            pltpu.SemaphoreType.DMA,
        ],
    )
    def sc_gather_sum(wl_ids, pos_ids, hop_ids, wl_t, pos_p, hop_p, out,
                      iwl, ipos, ihop, posv, hopv,
                      gwl0, gwl1, ob0, ob1,
                      gsem0, gsem1, wsem0, wsem1):
        gwl = (gwl0, gwl1)
        ob = (ob0, ob1)
        gsem = (gsem0, gsem1)
        wsem = (wsem0, wsem1)

        wid = lax.axis_index("s") * NC + lax.axis_index("c")
        wbase = pl.multiple_of(wid * ROWS_PER_W, ROWS_PER_W)
        gbase = pl.multiple_of(h * NHALF + wid * ROWS_PER_W, ROWS_PER_W)

        pltpu.sync_copy(wl_ids.at[pl.ds(gbase, ROWS_PER_W)], iwl)
        pltpu.sync_copy(pos_ids.at[pl.ds(gbase, ROWS_PER_W)], ipos)
        pltpu.sync_copy(hop_ids.at[pl.ds(gbase, ROWS_PER_W)], ihop)
        pltpu.sync_copy(pos_p, posv)
        pltpu.sync_copy(hop_p, hopv)

        def start_gather(s, off):
            pltpu.async_copy(wl_t.at[iwl.at[pl.ds(off, R)]], gwl[s], gsem[s])

        def wait_gather(s):
            pltpu.make_async_copy(wl_t.at[pl.ds(0, R)], gwl[s], gsem[s]).wait()

        def add_pass(s, off):
            gb = gwl[s]
            obs = ob[s]
            himask = jnp.int32(-65536)

            def grp_body(k, gcarry):
                rb = k * 16
                pid16 = ipos[pl.ds(off + rb, 16)]
                hid16 = ihop[pl.ds(off + rb, 16)]
                for j in range(16):
                    r = rb + j
                    selj = jnp.full((16,), j, jnp.int32)
                    prow = pid16.at[selj].get(mode="promise_in_bounds") << 6
                    hrow = hid16.at[selj].get(mode="promise_in_bounds") << 6
                    for cg in range(4):
                        colv = lax.iota(jnp.int32, 16) + (16 * cg)
                        pp = plsc.load_gather(posv, [prow + colv])
                        hp = plsc.load_gather(hopv, [hrow + colv])
                        plo = plsc.bitcast(pp << 16, jnp.float32)
                        phi = plsc.bitcast(pp & himask, jnp.float32)
                        hlo = plsc.bitcast(hp << 16, jnp.float32)
                        hhi = plsc.bitcast(hp & himask, jnp.float32)
                        slo = pl.ds(32 * cg, 16)
                        shi = pl.ds(32 * cg + 16, 16)
                        obs[r, slo] = gb[r, slo] + plo + hlo
                        obs[r, shi] = gb[r, shi] + phi + hhi
                return gcarry

            lax.fori_loop(0, R // 16, grp_body, 0)

        def start_write(s, off):
            pltpu.async_copy(ob[s], out.at[pl.ds(wbase + off, R)], wsem[s])

        def wait_write(s):
            pltpu.make_async_copy(ob[s], out.at[pl.ds(0, R)], wsem[s]).wait()

        start_gather(0, 0)
        start_gather(1, R)

        def pair_body(p, carry):
            off0 = p * (2 * R)

            @pl.when(p > 0)
            def _():
                wait_write(0)
            wait_gather(0)
            add_pass(0, off0)
            start_write(0, off0)

            @pl.when(p < NPAIR - 1)
            def _():
                start_gather(0, off0 + 2 * R)

            @pl.when(p > 0)
            def _():
                wait_write(1)
            wait_gather(1)
            add_pass(1, off0 + R)
            start_write(1, off0 + R)

            @pl.when(p < NPAIR - 1)
            def _():
                start_gather(1, off0 + 3 * R)

            return carry

        lax.fori_loop(0, NPAIR, pair_body, 0)
        wait_write(0)
        wait_write(1)

    return sc_gather_sum


_sc_gather_sum = tuple(_sc_gather_sum_build(h) for h in range(NSPLIT))


TBB = 64
TROWS = TBB * S
NBLK = B // TBB
NBLK_H = NBLK // NSPLIT


def _tc_finish_body(raw_ref, w_ref, b_ref, g_ref, gamma_ref, beta_ref,
                    part_ref, out_ref):
    del part_ref
    x3 = raw_ref[...]
    x = jnp.dot(x3.reshape(TROWS, FEAT), w_ref[...],
                preferred_element_type=jnp.float32)
    x = x + b_ref[...] + g_ref[...]
    mu = jnp.mean(x, axis=1, keepdims=True)
    d = x - mu
    var = jnp.mean(d * d, axis=1, keepdims=True)
    y = d * lax.rsqrt(var + EPS)
    y = y * gamma_ref[...] + beta_ref[...]
    out_ref[...] = y.reshape(TBB, S, HID)


def _tc_finish_build(h):
    off = h * NBLK_H
    return pl.pallas_call(
        _tc_finish_body,
        grid=(NBLK_H,),
        in_specs=[
            pl.BlockSpec((TBB, S, FEAT), lambda i: (i + off, 0, 0)),
            pl.BlockSpec((FEAT, HID), lambda i: (0, 0)),
            pl.BlockSpec((1, HID), lambda i: (0, 0)),
            pl.BlockSpec((TROWS, HID), lambda i: (i, 0)),
            pl.BlockSpec((1, HID), lambda i: (0, 0)),
            pl.BlockSpec((1, HID), lambda i: (0, 0)),
            pl.BlockSpec(memory_space=pl.ANY),
        ],
        out_specs=pl.BlockSpec((TBB, S, HID), lambda i: (i + off, 0, 0)),
        out_shape=jax.ShapeDtypeStruct((B, S, HID), jnp.float32),
        input_output_aliases={6: 0},
    )


_tc_finish = tuple(_tc_finish_build(h) for h in range(NSPLIT))


def kernel(raw_features, wl_role_ids, init_pos_ids, hop_dis_ids, W, b,
           wl_table, pos_table, hop_table, ln_gamma, ln_beta):
    wl_ids = wl_role_ids.reshape(N)
    pos_ids = init_pos_ids.reshape(N)
    hop_ids = hop_dis_ids.reshape(N)
    pos_p = _pack_bf16_lohi(pos_table)
    hop_p = _pack_bf16_lohi(hop_table)
    gs = [f(wl_ids, pos_ids, hop_ids, wl_table, pos_p, hop_p)
          for f in _sc_gather_sum]
    b2 = b.reshape(1, HID)
    gm = ln_gamma.reshape(1, HID)
    bt = ln_beta.reshape(1, HID)
    out = jnp.zeros((B, S, HID), jnp.float32)
    for h in range(NSPLIT):
        out = _tc_finish[h](raw_features, W, b2, gs[h], gm, bt, out)
    return out


# speedup vs baseline: 1.1450x; 1.0091x over previous
import functools

import jax
import jax.numpy as jnp
from jax import lax
from jax.experimental import pallas as pl
from jax.experimental.pallas import tpu as pltpu
from jax.experimental.pallas import tpu_sc as plsc

B, S = 4096, 50
FEAT, HID = 128, 128
N = B * S
EPS = 1e-12

NC, NS = 2, 16
NW = NC * NS
NSPLIT = 2
NHALF = N // NSPLIT
ROWS_PER_W = NHALF // NW
R = 80
NCHUNK = ROWS_PER_W // R
NPAIR = NCHUNK // 2


def _pack_bf16_lohi(t):
    u = lax.bitcast_convert_type(t, jnp.uint32)
    bf = (u + jnp.uint32(0x7FFF) + ((u >> 16) & jnp.uint32(1))) >> 16
    v = bf.reshape(t.shape[0], 4, 2, 16)
    packed = (v[:, :, 1, :] << 16) | v[:, :, 0, :]
    return lax.bitcast_convert_type(packed.reshape(t.shape[0] * 64), jnp.int32)


POSV, HOPV = 1000, 100


def _sc_gather_sum_build(h):
    mesh = plsc.VectorSubcoreMesh(core_axis_name="c", subcore_axis_name="s")

    @functools.partial(
        pl.kernel,
        mesh=mesh,
        compiler_params=pltpu.CompilerParams(needs_layout_passes=False),
        out_type=jax.ShapeDtypeStruct((NHALF, HID), jnp.float32),
        scratch_types=[
            pltpu.VMEM((ROWS_PER_W,), jnp.int32),
            pltpu.VMEM((ROWS_PER_W,), jnp.int32),
            pltpu.VMEM((ROWS_PER_W,), jnp.int32),
            pltpu.VMEM((POSV * HID // 2,), jnp.int32),
            pltpu.VMEM((HOPV * HID // 2,), jnp.int32),
            pltpu.VMEM((R, HID), jnp.float32),
            pltpu.VMEM((R, HID), jnp.float32),
            pltpu.VMEM((R, HID), jnp.float32),
            pltpu.VMEM((R, HID), jnp.float32),
            pltpu.SemaphoreType.DMA,
            pltpu.SemaphoreType.DMA,
            pltpu.SemaphoreType.DMA,
            pltpu.SemaphoreType.DMA,
        ],
    )
    def sc_gather_sum(wl_ids, pos_ids, hop_ids, wl_t, pos_p, hop_p, out,
                      iwl, ipos, ihop, posv, hopv,
                      gwl0, gwl1, ob0, ob1,
                      gsem0, gsem1, wsem0, wsem1):
        gwl = (gwl0, gwl1)
        ob = (ob0, ob1)
        gsem = (gsem0, gsem1)
        wsem = (wsem0, wsem1)

        wid = lax.axis_index("s") * NC + lax.axis_index("c")
        wbase = pl.multiple_of(wid * ROWS_PER_W, ROWS_PER_W)
        gbase = pl.multiple_of(h * NHALF + wid * ROWS_PER_W, ROWS_PER_W)

        pltpu.sync_copy(wl_ids.at[pl.ds(gbase, ROWS_PER_W)], iwl)
        pltpu.sync_copy(pos_ids.at[pl.ds(gbase, ROWS_PER_W)], ipos)
        pltpu.sync_copy(hop_ids.at[pl.ds(gbase, ROWS_PER_W)], ihop)
        pltpu.sync_copy(pos_p, posv)
        pltpu.sync_copy(hop_p, hopv)

        def start_gather(s, off):
            pltpu.async_copy(wl_t.at[iwl.at[pl.ds(off, R)]], gwl[s], gsem[s])

        def wait_gather(s):
            pltpu.make_async_copy(wl_t.at[pl.ds(0, R)], gwl[s], gsem[s]).wait()

        def add_pass(s, off):
            gb = gwl[s]
            obs = ob[s]
            himask = jnp.int32(-65536)

            def grp_body(k, gcarry):
                rb = k * 16
                pid16 = ipos[pl.ds(off + rb, 16)]
                hid16 = ihop[pl.ds(off + rb, 16)]
                for j in range(16):
                    r = rb + j
                    selj = jnp.full((16,), j, jnp.int32)
                    prow = pid16.at[selj].get(mode="promise_in_bounds") << 6
                    hrow = hid16.at[selj].get(mode="promise_in_bounds") << 6
                    for cg in range(4):
                        colv = lax.iota(jnp.int32, 16) + (16 * cg)
                        pp = plsc.load_gather(posv, [prow + colv])
                        hp = plsc.load_gather(hopv, [hrow + colv])
                        plo = plsc.bitcast(pp << 16, jnp.float32)
                        phi = plsc.bitcast(pp & himask, jnp.float32)
                        hlo = plsc.bitcast(hp << 16, jnp.float32)
                        hhi = plsc.bitcast(hp & himask, jnp.float32)
                        slo = pl.ds(32 * cg, 16)
                        shi = pl.ds(32 * cg + 16, 16)
                        obs[r, slo] = gb[r, slo] + plo + hlo
                        obs[r, shi] = gb[r, shi] + phi + hhi
                return gcarry

            lax.fori_loop(0, R // 16, grp_body, 0)

        def start_write(s, off):
            pltpu.async_copy(ob[s], out.at[pl.ds(wbase + off, R)], wsem[s])

        def wait_write(s):
            pltpu.make_async_copy(ob[s], out.at[pl.ds(0, R)], wsem[s]).wait()

        start_gather(0, 0)
        start_gather(1, R)

        def pair_body(p, carry):
            off0 = p * (2 * R)

            @pl.when(p > 0)
            def _():
                wait_write(0)
            wait_gather(0)
            add_pass(0, off0)
            start_write(0, off0)

            @pl.when(p < NPAIR - 1)
            def _():
                start_gather(0, off0 + 2 * R)

            @pl.when(p > 0)
            def _():
                wait_write(1)
            wait_gather(1)
            add_pass(1, off0 + R)
            start_write(1, off0 + R)

            @pl.when(p < NPAIR - 1)
            def _():
                start_gather(1, off0 + 3 * R)

            return carry

        lax.fori_loop(0, NPAIR, pair_body, 0)
        wait_write(0)
        wait_write(1)

    return sc_gather_sum


_sc_gather_sum = tuple(_sc_gather_sum_build(h) for h in range(NSPLIT))


TBB = 128
TROWS = TBB * S
NBLK = B // TBB
NBLK_H = NBLK // NSPLIT


def _tc_finish_body(raw_ref, w_ref, b_ref, g_ref, gamma_ref, beta_ref,
                    part_ref, out_ref):
    del part_ref
    x3 = raw_ref[...]
    x = jnp.dot(x3.reshape(TROWS, FEAT), w_ref[...],
                preferred_element_type=jnp.float32)
    x = x + b_ref[...] + g_ref[...]
    mu = jnp.mean(x, axis=1, keepdims=True)
    d = x - mu
    var = jnp.mean(d * d, axis=1, keepdims=True)
    y = d * lax.rsqrt(var + EPS)
    y = y * gamma_ref[...] + beta_ref[...]
    out_ref[...] = y.reshape(TBB, S, HID)


def _tc_finish_build(h):
    off = h * NBLK_H
    return pl.pallas_call(
        _tc_finish_body,
        grid=(NBLK_H,),
        in_specs=[
            pl.BlockSpec((TBB, S, FEAT), lambda i: (i + off, 0, 0)),
            pl.BlockSpec((FEAT, HID), lambda i: (0, 0)),
            pl.BlockSpec((1, HID), lambda i: (0, 0)),
            pl.BlockSpec((TROWS, HID), lambda i: (i, 0)),
            pl.BlockSpec((1, HID), lambda i: (0, 0)),
            pl.BlockSpec((1, HID), lambda i: (0, 0)),
            pl.BlockSpec(memory_space=pl.ANY),
        ],
        out_specs=pl.BlockSpec((TBB, S, HID), lambda i: (i + off, 0, 0)),
        out_shape=jax.ShapeDtypeStruct((B, S, HID), jnp.float32),
        input_output_aliases={6: 0},
    )


_tc_finish = tuple(_tc_finish_build(h) for h in range(NSPLIT))


def kernel(raw_features, wl_role_ids, init_pos_ids, hop_dis_ids, W, b,
           wl_table, pos_table, hop_table, ln_gamma, ln_beta):
    wl_ids = wl_role_ids.reshape(N)
    pos_ids = init_pos_ids.reshape(N)
    hop_ids = hop_dis_ids.reshape(N)
    pos_p = _pack_bf16_lohi(pos_table)
    hop_p = _pack_bf16_lohi(hop_table)
    gs = [f(wl_ids, pos_ids, hop_ids, wl_table, pos_p, hop_p)
          for f in _sc_gather_sum]
    b2 = b.reshape(1, HID)
    gm = ln_gamma.reshape(1, HID)
    bt = ln_beta.reshape(1, HID)
    out = jnp.zeros((B, S, HID), jnp.float32)
    for h in range(NSPLIT):
        out = _tc_finish[h](raw_features, W, b2, gs[h], gm, bt, out)
    return out


# speedup vs baseline: 1.1469x; 1.0017x over previous
import functools

import jax
import jax.numpy as jnp
from jax import lax
from jax.experimental import pallas as pl
from jax.experimental.pallas import tpu as pltpu
from jax.experimental.pallas import tpu_sc as plsc

B, S = 4096, 50
FEAT, HID = 128, 128
N = B * S
EPS = 1e-12

NC, NS = 2, 16
NW = NC * NS
NSPLIT = 2
NHALF = N // NSPLIT
ROWS_PER_W = NHALF // NW
R = 80
NCHUNK = ROWS_PER_W // R
NPAIR = NCHUNK // 2


def _pack_bf16_lohi(t):
    u = lax.bitcast_convert_type(t, jnp.uint32)
    bf = (u + jnp.uint32(0x7FFF) + ((u >> 16) & jnp.uint32(1))) >> 16
    v = bf.reshape(t.shape[0], 4, 2, 16)
    packed = (v[:, :, 1, :] << 16) | v[:, :, 0, :]
    return lax.bitcast_convert_type(packed.reshape(t.shape[0] * 64), jnp.int32)


POSV, HOPV = 1000, 100


def _sc_gather_sum_build(h):
    mesh = plsc.VectorSubcoreMesh(core_axis_name="c", subcore_axis_name="s")

    @functools.partial(
        pl.kernel,
        mesh=mesh,
        compiler_params=pltpu.CompilerParams(needs_layout_passes=False),
        out_type=jax.ShapeDtypeStruct((NHALF, HID), jnp.float32),
        scratch_types=[
            pltpu.VMEM((ROWS_PER_W,), jnp.int32),
            pltpu.VMEM((ROWS_PER_W,), jnp.int32),
            pltpu.VMEM((ROWS_PER_W,), jnp.int32),
            pltpu.VMEM((POSV * HID // 2,), jnp.int32),
            pltpu.VMEM((HOPV * HID // 2,), jnp.int32),
            pltpu.VMEM((R, HID), jnp.float32),
            pltpu.VMEM((R, HID), jnp.float32),
            pltpu.VMEM((R, HID), jnp.float32),
            pltpu.VMEM((R, HID), jnp.float32),
            pltpu.SemaphoreType.DMA,
            pltpu.SemaphoreType.DMA,
            pltpu.SemaphoreType.DMA,
            pltpu.SemaphoreType.DMA,
        ],
    )
    def sc_gather_sum(wl_ids, pos_ids, hop_ids, wl_t, pos_p, hop_p, out,
                      iwl, ipos, ihop, posv, hopv,
                      gwl0, gwl1, ob0, ob1,
                      gsem0, gsem1, wsem0, wsem1):
        gwl = (gwl0, gwl1)
        ob = (ob0, ob1)
        gsem = (gsem0, gsem1)
        wsem = (wsem0, wsem1)

        wid = lax.axis_index("s") * NC + lax.axis_index("c")
        wbase = pl.multiple_of(wid * ROWS_PER_W, ROWS_PER_W)
        gbase = pl.multiple_of(h * NHALF + wid * ROWS_PER_W, ROWS_PER_W)

        def start_gather(s, off):
            pltpu.async_copy(wl_t.at[iwl.at[pl.ds(off, R)]], gwl[s], gsem[s])

        def wait_gather(s):
            pltpu.make_async_copy(wl_t.at[pl.ds(0, R)], gwl[s], gsem[s]).wait()

        def add_pass(s, off):
            gb = gwl[s]
            obs = ob[s]
            himask = jnp.int32(-65536)

            def grp_body(k, gcarry):
                rb = k * 16
                pid16 = ipos[pl.ds(off + rb, 16)]
                hid16 = ihop[pl.ds(off + rb, 16)]
                for j in range(16):
                    r = rb + j
                    selj = jnp.full((16,), j, jnp.int32)
                    prow = pid16.at[selj].get(mode="promise_in_bounds") << 6
                    hrow = hid16.at[selj].get(mode="promise_in_bounds") << 6
                    for cg in range(4):
                        colv = lax.iota(jnp.int32, 16) + (16 * cg)
                        pp = plsc.load_gather(posv, [prow + colv])
                        hp = plsc.load_gather(hopv, [hrow + colv])
                        plo = plsc.bitcast(pp << 16, jnp.float32)
                        phi = plsc.bitcast(pp & himask, jnp.float32)
                        hlo = plsc.bitcast(hp << 16, jnp.float32)
                        hhi = plsc.bitcast(hp & himask, jnp.float32)
                        slo = pl.ds(32 * cg, 16)
                        shi = pl.ds(32 * cg + 16, 16)
                        obs[r, slo] = gb[r, slo] + plo + hlo
                        obs[r, shi] = gb[r, shi] + phi + hhi
                return gcarry

            lax.fori_loop(0, R // 16, grp_body, 0)

        def start_write(s, off):
            pltpu.async_copy(ob[s], out.at[pl.ds(wbase + off, R)], wsem[s])

        def wait_write(s):
            pltpu.make_async_copy(ob[s], out.at[pl.ds(0, R)], wsem[s]).wait()

        pltpu.sync_copy(wl_ids.at[pl.ds(gbase, ROWS_PER_W)], iwl)
        start_gather(0, 0)
        start_gather(1, R)
        pltpu.sync_copy(pos_ids.at[pl.ds(gbase, ROWS_PER_W)], ipos)
        pltpu.sync_copy(hop_ids.at[pl.ds(gbase, ROWS_PER_W)], ihop)
        pltpu.sync_copy(pos_p, posv)
        pltpu.sync_copy(hop_p, hopv)

        def pair_body(p, carry):
            off0 = p * (2 * R)

            @pl.when(p > 0)
            def _():
                wait_write(0)
            wait_gather(0)
            add_pass(0, off0)
            start_write(0, off0)

            @pl.when(p < NPAIR - 1)
            def _():
                start_gather(0, off0 + 2 * R)

            @pl.when(p > 0)
            def _():
                wait_write(1)
            wait_gather(1)
            add_pass(1, off0 + R)
            start_write(1, off0 + R)

            @pl.when(p < NPAIR - 1)
            def _():
                start_gather(1, off0 + 3 * R)

            return carry

        lax.fori_loop(0, NPAIR, pair_body, 0)
        wait_write(0)
        wait_write(1)

    return sc_gather_sum


_sc_gather_sum = tuple(_sc_gather_sum_build(h) for h in range(NSPLIT))


TBB = 128
TROWS = TBB * S
NBLK = B // TBB
NBLK_H = NBLK // NSPLIT


def _tc_finish_body(raw_ref, w_ref, b_ref, g_ref, gamma_ref, beta_ref,
                    part_ref, out_ref):
    del part_ref
    x3 = raw_ref[...]
    x = jnp.dot(x3.reshape(TROWS, FEAT), w_ref[...],
                preferred_element_type=jnp.float32)
    x = x + b_ref[...] + g_ref[...]
    mu = jnp.mean(x, axis=1, keepdims=True)
    d = x - mu
    var = jnp.mean(d * d, axis=1, keepdims=True)
    y = d * lax.rsqrt(var + EPS)
    y = y * gamma_ref[...] + beta_ref[...]
    out_ref[...] = y.reshape(TBB, S, HID)


def _tc_finish_build(h):
    off = h * NBLK_H
    return pl.pallas_call(
        _tc_finish_body,
        grid=(NBLK_H,),
        in_specs=[
            pl.BlockSpec((TBB, S, FEAT), lambda i: (i + off, 0, 0)),
            pl.BlockSpec((FEAT, HID), lambda i: (0, 0)),
            pl.BlockSpec((1, HID), lambda i: (0, 0)),
            pl.BlockSpec((TROWS, HID), lambda i: (i, 0)),
            pl.BlockSpec((1, HID), lambda i: (0, 0)),
            pl.BlockSpec((1, HID), lambda i: (0, 0)),
            pl.BlockSpec(memory_space=pl.ANY),
        ],
        out_specs=pl.BlockSpec((TBB, S, HID), lambda i: (i + off, 0, 0)),
        out_shape=jax.ShapeDtypeStruct((B, S, HID), jnp.float32),
        input_output_aliases={6: 0},
    )


_tc_finish = tuple(_tc_finish_build(h) for h in range(NSPLIT))


def kernel(raw_features, wl_role_ids, init_pos_ids, hop_dis_ids, W, b,
           wl_table, pos_table, hop_table, ln_gamma, ln_beta):
    wl_ids = wl_role_ids.reshape(N)
    pos_ids = init_pos_ids.reshape(N)
    hop_ids = hop_dis_ids.reshape(N)
    pos_p = _pack_bf16_lohi(pos_table)
    hop_p = _pack_bf16_lohi(hop_table)
    gs = [f(wl_ids, pos_ids, hop_ids, wl_table, pos_p, hop_p)
          for f in _sc_gather_sum]
    b2 = b.reshape(1, HID)
    gm = ln_gamma.reshape(1, HID)
    bt = ln_beta.reshape(1, HID)
    out = jnp.zeros((B, S, HID), jnp.float32)
    for h in range(NSPLIT):
        out = _tc_finish[h](raw_features, W, b2, gs[h], gm, bt, out)
    return out


# speedup vs baseline: 1.1524x; 1.0048x over previous
import functools

import jax
import jax.numpy as jnp
from jax import lax
from jax.experimental import pallas as pl
from jax.experimental.pallas import tpu as pltpu
from jax.experimental.pallas import tpu_sc as plsc

B, S = 4096, 50
FEAT, HID = 128, 128
N = B * S
EPS = 1e-12

NC, NS = 2, 16
NW = NC * NS
NSPLIT = 2
NHALF = N // NSPLIT
ROWS_PER_W = NHALF // NW
R = 80
NCHUNK = ROWS_PER_W // R
NPAIR = NCHUNK // 2


def _pack_bf16_lohi(t):
    u = lax.bitcast_convert_type(t, jnp.uint32)
    bf = (u + jnp.uint32(0x7FFF) + ((u >> 16) & jnp.uint32(1))) >> 16
    v = bf.reshape(t.shape[0], 4, 2, 16)
    packed = (v[:, :, 1, :] << 16) | v[:, :, 0, :]
    return lax.bitcast_convert_type(packed.reshape(t.shape[0] * 64), jnp.int32)


POSV, HOPV = 1000, 100


def _sc_gather_sum_build(h):
    mesh = plsc.VectorSubcoreMesh(core_axis_name="c", subcore_axis_name="s")

    @functools.partial(
        pl.kernel,
        mesh=mesh,
        compiler_params=pltpu.CompilerParams(needs_layout_passes=False),
        out_type=jax.ShapeDtypeStruct((NHALF, HID), jnp.float32),
        scratch_types=[
            pltpu.VMEM((ROWS_PER_W,), jnp.int32),
            pltpu.VMEM((ROWS_PER_W,), jnp.int32),
            pltpu.VMEM((ROWS_PER_W,), jnp.int32),
            pltpu.VMEM((POSV * HID // 2,), jnp.int32),
            pltpu.VMEM((HOPV * HID // 2,), jnp.int32),
            pltpu.VMEM((R, HID), jnp.float32),
            pltpu.VMEM((R, HID), jnp.float32),
            pltpu.VMEM((R, HID), jnp.float32),
            pltpu.VMEM((R, HID), jnp.float32),
            pltpu.SemaphoreType.DMA,
            pltpu.SemaphoreType.DMA,
            pltpu.SemaphoreType.DMA,
            pltpu.SemaphoreType.DMA,
        ],
    )
    def sc_gather_sum(wl_ids, pos_ids, hop_ids, wl_t, pos_p, hop_p, out,
                      iwl, ipos, ihop, posv, hopv,
                      gwl0, gwl1, ob0, ob1,
                      gsem0, gsem1, wsem0, wsem1):
        gwl = (gwl0, gwl1)
        ob = (ob0, ob1)
        gsem = (gsem0, gsem1)
        wsem = (wsem0, wsem1)

        wid = lax.axis_index("s") * NC + lax.axis_index("c")
        wbase = pl.multiple_of(wid * ROWS_PER_W, ROWS_PER_W)
        gbase = pl.multiple_of(h * NHALF + wid * ROWS_PER_W, ROWS_PER_W)

        def start_gather(s, off):
            pltpu.async_copy(wl_t.at[iwl.at[pl.ds(off, R)]], gwl[s], gsem[s])

        def wait_gather(s):
            pltpu.make_async_copy(wl_t.at[pl.ds(0, R)], gwl[s], gsem[s]).wait()

        def add_pass(s, off):
            gb = gwl[s]
            obs = ob[s]
            himask = jnp.int32(-65536)

            def grp_body(k, gcarry):
                rb = k * 16
                pid16 = ipos[pl.ds(off + rb, 16)]
                hid16 = ihop[pl.ds(off + rb, 16)]
                for j in range(16):
                    r = rb + j
                    selj = jnp.full((16,), j, jnp.int32)
                    prow = pid16.at[selj].get(mode="promise_in_bounds") << 6
                    hrow = hid16.at[selj].get(mode="promise_in_bounds") << 6
                    for cg in range(4):
                        colv = lax.iota(jnp.int32, 16) + (16 * cg)
                        pp = plsc.load_gather(posv, [prow + colv])
                        hp = plsc.load_gather(hopv, [hrow + colv])
                        plo = plsc.bitcast(pp << 16, jnp.float32)
                        phi = plsc.bitcast(pp & himask, jnp.float32)
                        hlo = plsc.bitcast(hp << 16, jnp.float32)
                        hhi = plsc.bitcast(hp & himask, jnp.float32)
                        slo = pl.ds(32 * cg, 16)
                        shi = pl.ds(32 * cg + 16, 16)
                        obs[r, slo] = gb[r, slo] + plo + hlo
                        obs[r, shi] = gb[r, shi] + phi + hhi
                return gcarry

            lax.fori_loop(0, R // 16, grp_body, 0)

        def start_write(s, off):
            pltpu.async_copy(ob[s], out.at[pl.ds(wbase + off, R)], wsem[s])

        def wait_write(s):
            pltpu.make_async_copy(ob[s], out.at[pl.ds(0, R)], wsem[s]).wait()

        pltpu.sync_copy(wl_ids.at[pl.ds(gbase, ROWS_PER_W)], iwl)
        start_gather(0, 0)
        start_gather(1, R)
        pltpu.sync_copy(pos_ids.at[pl.ds(gbase, ROWS_PER_W)], ipos)
        pltpu.sync_copy(hop_ids.at[pl.ds(gbase, ROWS_PER_W)], ihop)
        pltpu.sync_copy(pos_p, posv)
        pltpu.sync_copy(hop_p, hopv)

        def pair_body(p, carry):
            off0 = p * (2 * R)

            @pl.when(p > 0)
            def _():
                wait_write(0)
            wait_gather(0)
            add_pass(0, off0)
            start_write(0, off0)

            @pl.when(p < NPAIR - 1)
            def _():
                start_gather(0, off0 + 2 * R)

            @pl.when(p > 0)
            def _():
                wait_write(1)
            wait_gather(1)
            add_pass(1, off0 + R)
            start_write(1, off0 + R)

            @pl.when(p < NPAIR - 1)
            def _():
                start_gather(1, off0 + 3 * R)

            return carry

        lax.fori_loop(0, NPAIR, pair_body, 0)
        wait_write(0)
        wait_write(1)

    return sc_gather_sum


_sc_gather_sum = tuple(_sc_gather_sum_build(h) for h in range(NSPLIT))


TBB = 256
TROWS = TBB * S
NBLK = B // TBB
NBLK_H = NBLK // NSPLIT


def _tc_finish_body(raw_ref, w_ref, b_ref, g_ref, gamma_ref, beta_ref,
                    part_ref, out_ref):
    del part_ref
    x3 = raw_ref[...]
    x = jnp.dot(x3.reshape(TROWS, FEAT), w_ref[...],
                preferred_element_type=jnp.float32)
    x = x + b_ref[...] + g_ref[...]
    mu = jnp.mean(x, axis=1, keepdims=True)
    d = x - mu
    var = jnp.mean(d * d, axis=1, keepdims=True)
    y = d * lax.rsqrt(var + EPS)
    y = y * gamma_ref[...] + beta_ref[...]
    out_ref[...] = y.reshape(TBB, S, HID)


def _tc_finish_build(h):
    off = h * NBLK_H
    return pl.pallas_call(
        _tc_finish_body,
        grid=(NBLK_H,),
        in_specs=[
            pl.BlockSpec((TBB, S, FEAT), lambda i: (i + off, 0, 0)),
            pl.BlockSpec((FEAT, HID), lambda i: (0, 0)),
            pl.BlockSpec((1, HID), lambda i: (0, 0)),
            pl.BlockSpec((TROWS, HID), lambda i: (i, 0)),
            pl.BlockSpec((1, HID), lambda i: (0, 0)),
            pl.BlockSpec((1, HID), lambda i: (0, 0)),
            pl.BlockSpec(memory_space=pl.ANY),
        ],
        out_specs=pl.BlockSpec((TBB, S, HID), lambda i: (i + off, 0, 0)),
        out_shape=jax.ShapeDtypeStruct((B, S, HID), jnp.float32),
        input_output_aliases={6: 0},
    )


_tc_finish = tuple(_tc_finish_build(h) for h in range(NSPLIT))


def kernel(raw_features, wl_role_ids, init_pos_ids, hop_dis_ids, W, b,
           wl_table, pos_table, hop_table, ln_gamma, ln_beta):
    wl_ids = wl_role_ids.reshape(N)
    pos_ids = init_pos_ids.reshape(N)
    hop_ids = hop_dis_ids.reshape(N)
    pos_p = _pack_bf16_lohi(pos_table)
    hop_p = _pack_bf16_lohi(hop_table)
    gs = [f(wl_ids, pos_ids, hop_ids, wl_table, pos_p, hop_p)
          for f in _sc_gather_sum]
    b2 = b.reshape(1, HID)
    gm = ln_gamma.reshape(1, HID)
    bt = ln_beta.reshape(1, HID)
    out = jnp.zeros((B, S, HID), jnp.float32)
    for h in range(NSPLIT):
        out = _tc_finish[h](raw_features, W, b2, gs[h], gm, bt, out)
    return out
